# baseline (speedup 1.0000x reference)
import jax
import jax.numpy as jnp
from jax.experimental import pallas as pl
from jax.experimental.pallas import tpu as pltpu

B = 2048
D = 768
H = 512
E = 8
K = 2
NQ = 4
QS = B // NQ


def _moe_kernel(x_hbm, wr_ref, br_ref, w1_ref, b1_ref, w2_ref, b2_ref,
                out_hbm, xf_ref, x16_ref, scale_ref, acc_ref, xsem, osem):
    e = pl.program_id(0)

    @pl.when(e == 0)
    def _prologue():
        for q in range(NQ):
            pltpu.make_async_copy(
                x_hbm.at[pl.ds(q * QS, QS), :],
                xf_ref.at[pl.ds(q * QS, QS), :], xsem.at[q]).start()
        for q in range(NQ):
            pltpu.make_async_copy(
                x_hbm.at[pl.ds(q * QS, QS), :],
                xf_ref.at[pl.ds(q * QS, QS), :], xsem.at[q]).wait()
            xq = xf_ref[pl.ds(q * QS, QS), :]
            logits = jax.lax.dot_general(
                xq, wr_ref[...], (((1,), (1,)), ((), ())),
                preferred_element_type=jnp.float32) + br_ref[...]
            m = jnp.max(logits, axis=-1, keepdims=True)
            ex = jnp.exp(logits - m)
            probs = ex / jnp.sum(ex, axis=-1, keepdims=True)
            eids = jax.lax.broadcasted_iota(jnp.int32, logits.shape, 1)
            i1 = jnp.argmax(logits, axis=-1, keepdims=True)
            masked = jnp.where(eids == i1, -jnp.inf, logits)
            i2 = jnp.argmax(masked, axis=-1, keepdims=True)
            sel = (eids == i1) | (eids == i2)
            scale_ref[pl.ds(q * QS, QS), :] = jnp.where(sel, probs, 0.0)
            x16_ref[pl.ds(q * QS, QS), :] = xq.astype(jnp.bfloat16)
        acc_ref[...] = jax.lax.dot_general(
            scale_ref[...], b2_ref[...], (((1,), (0,)), ((), ())),
            preferred_element_type=jnp.float32)

    x16 = x16_ref[...]
    sc = scale_ref[...]
    cols = jax.lax.broadcasted_iota(jnp.int32, sc.shape, 1)
    se = jnp.sum(jnp.where(cols == e, sc, 0.0), axis=1, keepdims=True)
    w1e = w1_ref[0].astype(jnp.bfloat16)
    w2e = w2_ref[0].astype(jnp.bfloat16)
    h = jax.lax.dot_general(
        x16, w1e, (((1,), (1,)), ((), ())),
        preferred_element_type=jnp.float32) + b1_ref[0]
    h = jnp.maximum(h, 0.0)
    h16 = (h * se).astype(jnp.bfloat16)

    @pl.when(e < E - 1)
    def _accumulate():
        acc_ref[...] += jax.lax.dot_general(
            h16, w2e, (((1,), (1,)), ((), ())),
            preferred_element_type=jnp.float32)

    @pl.when(e == E - 1)
    def _finalize():
        HB = B // 2
        for i in range(2):
            rows = pl.ds(i * HB, HB)
            acc_ref[rows, :] += jax.lax.dot_general(
                h16[i * HB:(i + 1) * HB], w2e, (((1,), (1,)), ((), ())),
                preferred_element_type=jnp.float32)
            pltpu.make_async_copy(
                acc_ref.at[rows, :], out_hbm.at[rows, :], osem.at[i]).start()
        for i in range(2):
            rows = pl.ds(i * HB, HB)
            pltpu.make_async_copy(
                acc_ref.at[rows, :], out_hbm.at[rows, :], osem.at[i]).wait()


def kernel(x, Wr, br, W1, b1, W2, b2):
    br2 = br.reshape(1, E)
    b13 = b1.reshape(E, 1, H)
    out = pl.pallas_call(
        _moe_kernel,
        grid=(E,),
        in_specs=[
            pl.BlockSpec(memory_space=pltpu.MemorySpace.HBM),
            pl.BlockSpec((E, D), lambda e: (0, 0)),
            pl.BlockSpec((1, E), lambda e: (0, 0)),
            pl.BlockSpec((1, H, D), lambda e: (e, 0, 0)),
            pl.BlockSpec((1, 1, H), lambda e: (e, 0, 0)),
            pl.BlockSpec((1, D, H), lambda e: (e, 0, 0)),
            pl.BlockSpec((E, D), lambda e: (0, 0)),
        ],
        out_specs=pl.BlockSpec(memory_space=pltpu.MemorySpace.HBM),
        out_shape=jax.ShapeDtypeStruct((B, D), jnp.float32),
        scratch_shapes=[
            pltpu.MemorySpace.VMEM((B, D), jnp.float32),
            pltpu.MemorySpace.VMEM((B, D), jnp.bfloat16),
            pltpu.MemorySpace.VMEM((B, E), jnp.float32),
            pltpu.MemorySpace.VMEM((B, D), jnp.float32),
            pltpu.SemaphoreType.DMA((NQ,)),
            pltpu.SemaphoreType.DMA((2,)),
        ],
    )(x, Wr, br2, W1, b13, W2, b2)
    return out
